# Initial kernel scaffold; baseline (speedup 1.0000x reference)
import jax
import jax.numpy as jnp
from jax.experimental import pallas as pl


def kernel(class_logits, box_regression, quad_box_regression, proposal_boxes):
    raise NotImplementedError("write your pallas kernel here")


# speedup vs baseline: 694.6030x; 694.6030x over previous
import functools

import jax
import jax.numpy as jnp
import numpy as np
from jax import lax
from jax.experimental import pallas as pl
from jax.experimental.pallas import tpu as pltpu
from jax.experimental.pallas import tpu_sc as plsc

N = 5000
C = 81
NCLS = C - 1
NP = 5120
K = 512
CBUF = K + 48
SEL = 128
SELBUF = SEL + 32
DETS = 100
DETS_PAD = 112
IMG_W = 1333.0
IMG_H = 800.0
SCORE_THRESH = 0.05
NMS_THRESH = 0.5
BBOX_XFORM_CLIP = float(np.log(1000.0 / 16.0))
NEG = -1e30


def _scalar(v):
  return jnp.max(v)


def _dense_body(ltf, ltn, dxr, dyr, dwr, dhr, pbr,
                sc_o, x1_o, y1_o, x2_o, y2_o):
  lt = ltf[...]
  m = jnp.max(lt, axis=0, keepdims=True)
  den = jnp.sum(jnp.exp(lt - m), axis=0, keepdims=True)
  sc_o[...] = jnp.exp(ltn[...] - m) / den

  pb = pbr[...]
  px1 = pb[0:1]
  py1 = pb[1:2]
  px2 = pb[2:3]
  py2 = pb[3:4]
  w = px2 - px1 + 1.0
  h = py2 - py1 + 1.0
  cx = px1 + 0.5 * w
  cy = py1 + 0.5 * h
  dx = dxr[...] / 10.0
  dy = dyr[...] / 10.0
  dw = jnp.minimum(dwr[...] / 5.0, BBOX_XFORM_CLIP)
  dh = jnp.minimum(dhr[...] / 5.0, BBOX_XFORM_CLIP)
  pcx = dx * w + cx
  pcy = dy * h + cy
  pw = jnp.exp(dw) * w
  ph = jnp.exp(dh) * h
  x1_o[...] = jnp.clip(pcx - 0.5 * pw, 0.0, IMG_W - 1.0)
  y1_o[...] = jnp.clip(pcy - 0.5 * ph, 0.0, IMG_H - 1.0)
  x2_o[...] = jnp.clip(pcx + 0.5 * pw - 1.0, 0.0, IMG_W - 1.0)
  y2_o[...] = jnp.clip(pcy + 0.5 * ph - 1.0, 0.0, IMG_H - 1.0)


def _dense_stage(lt_full, lt_nb, dxt, dyt, dwt, dht, pbt):
  LC = 1280
  grid = (NP // LC,)
  full = lambda r: pl.BlockSpec((r, LC), lambda i: (0, i))
  return pl.pallas_call(
      _dense_body,
      grid=grid,
      in_specs=[full(88), full(NCLS), full(NCLS), full(NCLS), full(NCLS),
                full(NCLS), full(8)],
      out_specs=[full(NCLS)] * 5,
      out_shape=[jax.ShapeDtypeStruct((NCLS, NP), jnp.float32)] * 5,
  )(lt_full, lt_nb, dxt, dyt, dwt, dht, pbt)


def _compact_body(sc_h, x1_h, y1_h, x2_h, y2_h,
                  scc_h, x1c_h, y1c_h, x2c_h, y2c_h, idxc_h,
                  srow, r1, r2, r3, r4, cs, c1, c2, c3, c4, ci):
  wid = lax.axis_index("s") * 2 + lax.axis_index("c")
  lane = lax.iota(jnp.int32, 16)

  def do_class(j):
    pltpu.sync_copy(sc_h.at[j], srow)
    pltpu.sync_copy(x1_h.at[j], r1)
    pltpu.sync_copy(y1_h.at[j], r2)
    pltpu.sync_copy(x2_h.at[j], r3)
    pltpu.sync_copy(y2_h.at[j], r4)

    def init(k, _):
      cs[pl.ds(k * 16, 16)] = jnp.full((16,), -1.0, jnp.float32)
      return 0
    lax.fori_loop(0, CBUF // 16, init, 0)

    def chunk(k, cnt):
      base = k * 16
      sv = srow[pl.ds(base, 16)]
      msk = (sv > SCORE_THRESH) & (base + lane < N)
      csum = plsc.cumsum(jnp.where(msk, 1, 0))
      off = jnp.minimum(cnt, K)
      pos = jnp.where(msk, off + csum - 1, K + 32 + lane)
      plsc.store_scatter(cs, [pos], sv)
      plsc.store_scatter(c1, [pos], r1[pl.ds(base, 16)])
      plsc.store_scatter(c2, [pos], r2[pl.ds(base, 16)])
      plsc.store_scatter(c3, [pos], r3[pl.ds(base, 16)])
      plsc.store_scatter(c4, [pos], r4[pl.ds(base, 16)])
      plsc.store_scatter(ci, [pos], base + lane)
      return cnt + _scalar(csum)

    lax.fori_loop(0, 313, chunk, jnp.int32(0))
    pltpu.sync_copy(cs.at[pl.ds(0, K)], scc_h.at[j])
    pltpu.sync_copy(c1.at[pl.ds(0, K)], x1c_h.at[j])
    pltpu.sync_copy(c2.at[pl.ds(0, K)], y1c_h.at[j])
    pltpu.sync_copy(c3.at[pl.ds(0, K)], x2c_h.at[j])
    pltpu.sync_copy(c4.at[pl.ds(0, K)], y2c_h.at[j])
    pltpu.sync_copy(ci.at[pl.ds(0, K)], idxc_h.at[j])

  for t in range(3):
    j = wid + t * 32
    @pl.when(j < NCLS)
    def _():
      do_class(j)


_SC_PARAMS = pltpu.CompilerParams(use_tc_tiling_on_sc=False,
                                  needs_layout_passes=False)


def _compact_stage(scores, x1, y1, x2, y2):
  mesh = plsc.VectorSubcoreMesh(core_axis_name="c", subcore_axis_name="s")
  f32 = jnp.float32
  kern = functools.partial(
      pl.kernel,
      mesh=mesh,
      out_type=[jax.ShapeDtypeStruct((NCLS, K), f32)] * 5
      + [jax.ShapeDtypeStruct((NCLS, K), jnp.int32)],
      scratch_types=[pltpu.VMEM((NP,), f32)] * 5
      + [pltpu.VMEM((CBUF,), f32)] * 5
      + [pltpu.VMEM((CBUF,), jnp.int32)],
      compiler_params=_SC_PARAMS,
  )(_compact_body)
  return kern(scores, x1, y1, x2, y2)


G = 8


def _nms_body(sc_r, x1_r, y1_r, x2_r, y2_r, kept_o):
  s = sc_r[...]
  x1 = x1_r[...]
  y1 = y1_r[...]
  x2 = x2_r[...]
  y2 = y2_r[...]
  valid = s > SCORE_THRESH
  area = (x2 - x1 + 1.0) * (y2 - y1 + 1.0)

  xx1 = jnp.maximum(x1[:, :, None], x1[:, None, :])
  yy1 = jnp.maximum(y1[:, :, None], y1[:, None, :])
  xx2 = jnp.minimum(x2[:, :, None], x2[:, None, :])
  yy2 = jnp.minimum(y2[:, :, None], y2[:, None, :])
  inter = (jnp.maximum(xx2 - xx1 + 1.0, 0.0)
           * jnp.maximum(yy2 - yy1 + 1.0, 0.0))
  iou = inter / (area[:, :, None] + area[:, None, :] - inter)

  pos = lax.broadcasted_iota(jnp.int32, (K, K), 0)
  posj = lax.broadcasted_iota(jnp.int32, (K, K), 1)
  better = ((s[:, None, :] > s[:, :, None])
            | ((s[:, None, :] == s[:, :, None]) & (posj < pos)[None]))
  msup = jnp.where(better & (iou > NMS_THRESH), 1.0, 0.0)

  keep0 = jnp.where(valid, 1.0, 0.0)

  def cond(c):
    return c[1]

  def body(c):
    keep, _ = c
    sup = lax.dot_general(msup, keep, (((2,), (1,)), ((0,), (0,))),
                          preferred_element_type=jnp.float32)
    new = jnp.where(valid & (sup < 0.5), 1.0, 0.0)
    return new, jnp.any(new != keep)

  keep, _ = lax.while_loop(cond, body, (keep0, jnp.bool_(True)))
  kept_o[...] = jnp.where(keep > 0.5, s, -1.0)


def _nms_stage(scc, x1c, y1c, x2c, y2c):
  spec = pl.BlockSpec((G, K), lambda i: (i, 0))
  return pl.pallas_call(
      _nms_body,
      grid=(NCLS // G,),
      in_specs=[spec] * 5,
      out_specs=spec,
      out_shape=jax.ShapeDtypeStruct((NCLS, K), jnp.float32),
  )(scc, x1c, y1c, x2c, y2c)


def _kth_body(kept_r, tau_o):
  s = kept_r[...]

  def body(_, c):
    lo, hi = c
    mid = (lo + hi) * 0.5
    cnt = jnp.sum(jnp.where(s > mid, 1, 0))
    take = cnt >= DETS
    return jnp.where(take, mid, lo), jnp.where(take, hi, mid)

  lo, _ = lax.fori_loop(0, 48, body, (jnp.float32(0.0), jnp.float32(1.01)))
  tau_o[...] = jnp.full((8, 128), lo, jnp.float32)


def _kth_stage(kept):
  return pl.pallas_call(
      _kth_body,
      out_shape=jax.ShapeDtypeStruct((8, 128), jnp.float32),
  )(kept)


def _final_body(kept_h, idxc_h, tau_h, br_h, qbr_h, pb_h,
                ob_h, oq_h, os_h, ol_h,
                kept_v, idxc_v, tau_v, pbf_v,
                sel_s, sel_c, sel_i,
                ord_s, ord_c, ord_i, ord_row,
                bi0, bi1, bi2, bi3, qi0, qi1, qi2, qi3, qi4, qi5, qi6, qi7,
                bv0, bv1, bv2, bv3, qv0, qv1, qv2, qv3, qv4, qv5, qv6, qv7,
                ob_v, oq_v, ol_v, sem):
  wid = lax.axis_index("s") * 2 + lax.axis_index("c")
  lane = lax.iota(jnp.int32, 16)

  @pl.when(wid == 0)
  def _():
    pltpu.sync_copy(kept_h, kept_v)
    pltpu.sync_copy(idxc_h, idxc_v)
    pltpu.sync_copy(tau_h.at[pl.ds(0, 16)], tau_v)
    pltpu.sync_copy(pb_h, pbf_v)
    tau = _scalar(tau_v[...])

    def init(k, _):
      sel_s[pl.ds(k * 16, 16)] = jnp.full((16,), NEG, jnp.float32)
      return 0
    lax.fori_loop(0, SELBUF // 16, init, 0)

    def init2(k, _):
      ord_s[pl.ds(k * 16, 16)] = jnp.zeros((16,), jnp.float32)
      ord_c[pl.ds(k * 16, 16)] = jnp.zeros((16,), jnp.int32)
      ord_i[pl.ds(k * 16, 16)] = jnp.zeros((16,), jnp.int32)
      return 0
    lax.fori_loop(0, (SEL + 16) // 16, init2, 0)

    def sel_chunk(g, cnt):
      cg = g // (K // 16)
      base = (g % (K // 16)) * 16
      sv = kept_v[cg, pl.ds(base, 16)]
      iv = idxc_v[cg, pl.ds(base, 16)]
      msk = sv > tau
      csum = plsc.cumsum(jnp.where(msk, 1, 0))
      off = jnp.minimum(cnt, SEL)
      pos = jnp.where(msk, off + csum - 1, SEL + 16 + lane)
      plsc.store_scatter(sel_s, [pos], sv)
      plsc.store_scatter(sel_c, [pos], jnp.full((16,), cg, jnp.int32))
      plsc.store_scatter(sel_i, [pos], iv)
      return cnt + _scalar(csum)

    lax.fori_loop(0, NCLS * (K // 16), sel_chunk, jnp.int32(0))

    def rank_one(a, _):
      ac = (a // 16) * 16
      al = a % 16
      lmask = lane == al
      av = sel_s[pl.ds(ac, 16)]
      sa = jnp.max(jnp.where(lmask, av, NEG))
      r = jnp.int32(0)
      for bc in range(SEL // 16):
        bv = sel_s[pl.ds(bc * 16, 16)]
        bpos = bc * 16 + lane
        better = (bv > sa) | ((bv == sa) & (bpos < a))
        r = r + _scalar(plsc.cumsum(jnp.where(better, 1, 0)))
      cv = sel_c[pl.ds(ac, 16)]
      iv = sel_i[pl.ds(ac, 16)]
      ca = _scalar(jnp.where(lmask, cv, 0))
      ia = _scalar(jnp.where(lmask, iv, 0))
      ridx = jnp.where(lmask, jnp.minimum(r, SEL - 1), SEL + lane)
      plsc.store_scatter(ord_s, [ridx], jnp.full((16,), sa, jnp.float32))
      plsc.store_scatter(ord_c, [ridx], jnp.full((16,), ca, jnp.int32))
      plsc.store_scatter(ord_i, [ridx], jnp.full((16,), ia, jnp.int32))
      return 0
    lax.fori_loop(0, SEL, rank_one, 0)

    bis = [bi0, bi1, bi2, bi3]
    qis = [qi0, qi1, qi2, qi3, qi4, qi5, qi6, qi7]
    bvs = [bv0, bv1, bv2, bv3]
    qvs = [qv0, qv1, qv2, qv3, qv4, qv5, qv6, qv7]

    def rowk(k, _):
      base = k * 16
      iv = ord_i[pl.ds(base, 16)]
      cv = ord_c[pl.ds(base, 16)]
      row = iv * C + cv + 1
      ord_row[pl.ds(base, 16)] = row
      for t in range(4):
        bis[t][pl.ds(base, 16)] = row * 4 + t
      for t in range(8):
        qis[t][pl.ds(base, 16)] = row * 8 + t
      return 0
    lax.fori_loop(0, SEL // 16, rowk, 0)

    copies = []
    for t in range(4):
      copies.append(pltpu.async_copy(br_h.at[bis[t]], bvs[t], sem))
    for t in range(8):
      copies.append(pltpu.async_copy(qbr_h.at[qis[t]], qvs[t], sem))
    for cp in copies:
      cp.wait()

    def slot_chunk(k, _):
      base = k * 16
      sl = base + lane
      i_v = ord_i[pl.ds(base, 16)]
      c_v = ord_c[pl.ds(base, 16)]
      s_v = ord_s[pl.ds(base, 16)]
      bx1 = plsc.load_gather(pbf_v, [i_v * 4])
      by1 = plsc.load_gather(pbf_v, [i_v * 4 + 1])
      bx2 = plsc.load_gather(pbf_v, [i_v * 4 + 2])
      by2 = plsc.load_gather(pbf_v, [i_v * 4 + 3])
      w = bx2 - bx1 + 1.0
      h = by2 - by1 + 1.0
      cx = bx1 + 0.5 * w
      cy = by1 + 0.5 * h
      d0 = bv0[pl.ds(base, 16)]
      d1 = bv1[pl.ds(base, 16)]
      d2 = bv2[pl.ds(base, 16)]
      d3 = bv3[pl.ds(base, 16)]
      dx = d0 / 10.0
      dy = d1 / 10.0
      dw = jnp.minimum(d2 / 5.0, BBOX_XFORM_CLIP)
      dh = jnp.minimum(d3 / 5.0, BBOX_XFORM_CLIP)
      pcx = dx * w + cx
      pcy = dy * h + cy
      pw = jnp.exp(dw) * w
      ph = jnp.exp(dh) * h
      ox1 = jnp.clip(pcx - 0.5 * pw, 0.0, IMG_W - 1.0)
      oy1 = jnp.clip(pcy - 0.5 * ph, 0.0, IMG_H - 1.0)
      ox2 = jnp.clip(pcx + 0.5 * pw - 1.0, 0.0, IMG_W - 1.0)
      oy2 = jnp.clip(pcy + 0.5 * ph - 1.0, 0.0, IMG_H - 1.0)
      plsc.store_scatter(ob_v, [sl * 8], ox1)
      plsc.store_scatter(ob_v, [sl * 8 + 1], oy1)
      plsc.store_scatter(ob_v, [sl * 8 + 2], ox2)
      plsc.store_scatter(ob_v, [sl * 8 + 3], oy2)
      for t in range(8):
        qt = qvs[t][pl.ds(base, 16)]
        if t % 2 == 0:
          ov = jnp.clip((qt / 10.0) * w + cx, 0.0, IMG_W - 1.0)
        else:
          ov = jnp.clip((qt / 10.0) * h + cy, 0.0, IMG_H - 1.0)
        plsc.store_scatter(oq_v, [sl * 8 + t], ov)
      ol_v[pl.ds(base, 16)] = jnp.where(s_v > 0.0, c_v + 1, 0)
      return 0
    lax.fori_loop(0, DETS_PAD // 16, slot_chunk, 0)

    pltpu.sync_copy(ob_v.at[pl.ds(0, DETS_PAD * 8)], ob_h)
    pltpu.sync_copy(oq_v.at[pl.ds(0, DETS_PAD * 8)], oq_h)
    pltpu.sync_copy(ord_s.at[pl.ds(0, DETS_PAD)], os_h)
    pltpu.sync_copy(ol_v.at[pl.ds(0, DETS_PAD)], ol_h)


def _selrank_body(kept_h, idxc_h, tau_h,
                  os_h, oc_h, oi_h,
                  kept_v, idxc_v, tau_v,
                  sel_s, sel_c, sel_i,
                  ord_s, ord_c, ord_i):
  wid = lax.axis_index("s") * 2 + lax.axis_index("c")
  lane = lax.iota(jnp.int32, 16)

  @pl.when(wid == 0)
  def _():
    pltpu.sync_copy(kept_h, kept_v)
    pltpu.sync_copy(idxc_h, idxc_v)
    pltpu.sync_copy(tau_h.at[pl.ds(0, 16)], tau_v)
    tau = _scalar(tau_v[...])

    def init(k, _):
      sel_s[pl.ds(k * 16, 16)] = jnp.full((16,), NEG, jnp.float32)
      return 0
    lax.fori_loop(0, SELBUF // 16, init, 0)

    def init2(k, _):
      ord_s[pl.ds(k * 16, 16)] = jnp.zeros((16,), jnp.float32)
      ord_c[pl.ds(k * 16, 16)] = jnp.zeros((16,), jnp.int32)
      ord_i[pl.ds(k * 16, 16)] = jnp.zeros((16,), jnp.int32)
      return 0
    lax.fori_loop(0, (SEL + 16) // 16, init2, 0)

    def sel_chunk(g, cnt):
      cg = g // (K // 16)
      base = (g % (K // 16)) * 16
      sv = kept_v[cg, pl.ds(base, 16)]
      iv = idxc_v[cg, pl.ds(base, 16)]
      msk = sv > tau
      csum = plsc.cumsum(jnp.where(msk, 1, 0))
      off = jnp.minimum(cnt, SEL)
      pos = jnp.where(msk, off + csum - 1, SEL + 16 + lane)
      plsc.store_scatter(sel_s, [pos], sv)
      plsc.store_scatter(sel_c, [pos], jnp.full((16,), cg, jnp.int32))
      plsc.store_scatter(sel_i, [pos], iv)
      return cnt + _scalar(csum)

    lax.fori_loop(0, NCLS * (K // 16), sel_chunk, jnp.int32(0))

    def rank_one(a, _):
      ac = (a // 16) * 16
      al = a % 16
      lmask = lane == al
      av = sel_s[pl.ds(ac, 16)]
      sa = jnp.max(jnp.where(lmask, av, NEG))
      r = jnp.int32(0)
      for bc in range(SEL // 16):
        bv = sel_s[pl.ds(bc * 16, 16)]
        bpos = bc * 16 + lane
        better = (bv > sa) | ((bv == sa) & (bpos < a))
        r = r + _scalar(plsc.cumsum(jnp.where(better, 1, 0)))
      cv = sel_c[pl.ds(ac, 16)]
      iv = sel_i[pl.ds(ac, 16)]
      ca = _scalar(jnp.where(lmask, cv, 0))
      ia = _scalar(jnp.where(lmask, iv, 0))
      ridx = jnp.where(lmask, jnp.minimum(r, SEL - 1), SEL + lane)
      plsc.store_scatter(ord_s, [ridx], jnp.full((16,), sa, jnp.float32))
      plsc.store_scatter(ord_c, [ridx], jnp.full((16,), ca, jnp.int32))
      plsc.store_scatter(ord_i, [ridx], jnp.full((16,), ia, jnp.int32))
      return 0
    lax.fori_loop(0, SEL, rank_one, 0)

    pltpu.sync_copy(ord_s.at[pl.ds(0, SEL)], os_h)
    pltpu.sync_copy(ord_c.at[pl.ds(0, SEL)], oc_h)
    pltpu.sync_copy(ord_i.at[pl.ds(0, SEL)], oi_h)


def _selrank_stage(kept, idxc, tau_flat):
  mesh = plsc.VectorSubcoreMesh(core_axis_name="c", subcore_axis_name="s")
  f32 = jnp.float32
  i32 = jnp.int32
  kern = functools.partial(
      pl.kernel,
      mesh=mesh,
      out_type=[
          jax.ShapeDtypeStruct((SEL,), f32),
          jax.ShapeDtypeStruct((SEL,), i32),
          jax.ShapeDtypeStruct((SEL,), i32),
      ],
      scratch_types=[
          pltpu.VMEM((NCLS, K), f32),
          pltpu.VMEM((NCLS, K), i32),
          pltpu.VMEM((16,), f32),
          pltpu.VMEM((SELBUF,), f32),
          pltpu.VMEM((SELBUF,), i32),
          pltpu.VMEM((SELBUF,), i32),
          pltpu.VMEM((SEL + 16,), f32),
          pltpu.VMEM((SEL + 16,), i32),
          pltpu.VMEM((SEL + 16,), i32),
      ],
      compiler_params=_SC_PARAMS,
  )(_selrank_body)
  return kern(kept, idxc, tau_flat)


def _final_stage(kept, idxc, tau_flat, br2, qbr2, pb):
  mesh = plsc.VectorSubcoreMesh(core_axis_name="c", subcore_axis_name="s")
  f32 = jnp.float32
  i32 = jnp.int32
  kern = functools.partial(
      pl.kernel,
      mesh=mesh,
      out_type=[
          jax.ShapeDtypeStruct((DETS_PAD * 8,), f32),
          jax.ShapeDtypeStruct((DETS_PAD * 8,), f32),
          jax.ShapeDtypeStruct((DETS_PAD,), f32),
          jax.ShapeDtypeStruct((DETS_PAD,), i32),
      ],
      scratch_types=[
          pltpu.VMEM((NCLS, K), f32),
          pltpu.VMEM((NCLS, K), i32),
          pltpu.VMEM((16,), f32),
          pltpu.VMEM((N * 4,), f32),
          pltpu.VMEM((SELBUF,), f32),
          pltpu.VMEM((SELBUF,), i32),
          pltpu.VMEM((SELBUF,), i32),
          pltpu.VMEM((SEL + 16,), f32),
          pltpu.VMEM((SEL + 16,), i32),
          pltpu.VMEM((SEL + 16,), i32),
          pltpu.VMEM((SEL,), i32),
      ]
      + [pltpu.VMEM((SEL,), i32)] * 12
      + [pltpu.VMEM((SEL,), f32)] * 12
      + [
          pltpu.VMEM((DETS_PAD * 8,), f32),
          pltpu.VMEM((DETS_PAD * 8,), f32),
          pltpu.VMEM((SEL,), i32),
          pltpu.SemaphoreType.DMA,
      ],
      compiler_params=_SC_PARAMS,
  )(_final_body)
  return kern(kept, idxc, tau_flat, br2, qbr2, pb)


def kernel(class_logits, box_regression, quad_box_regression, proposal_boxes):
  ltf = jnp.pad(class_logits.T, ((0, 88 - C), (0, NP - N)),
                constant_values=NEG)
  ltn = ltf[1:C]
  br3 = box_regression.reshape(N, C, 4)
  padp = lambda a: jnp.pad(a.T, ((0, 0), (0, NP - N)))
  dxt = padp(br3[:, 1:, 0])
  dyt = padp(br3[:, 1:, 1])
  dwt = padp(br3[:, 1:, 2])
  dht = padp(br3[:, 1:, 3])
  pbt = jnp.pad(proposal_boxes.T, ((0, 4), (0, NP - N)))

  scores, x1, y1, x2, y2 = _dense_stage(ltf, ltn, dxt, dyt, dwt, dht, pbt)
  scc, x1c, y1c, x2c, y2c, idxc = _compact_stage(scores, x1, y1, x2, y2)
  kept = _nms_stage(scc, x1c, y1c, x2c, y2c)
  tau = _kth_stage(kept).reshape(-1)
  brf = box_regression.reshape(N * C * 4)
  qbrf = quad_box_regression.reshape(N * C * 8)
  pbf = proposal_boxes.reshape(N * 4)
  obf, oqf, osc, olb = _final_stage(kept, idxc, tau, brf, qbrf, pbf)
  ob = obf.reshape(DETS_PAD, 8)
  oq = oqf.reshape(DETS_PAD, 8)
  return (ob[:DETS, :4], oq[:DETS], osc[:DETS], olb[:DETS])
